# Initial kernel scaffold; baseline (speedup 1.0000x reference)
import jax
import jax.numpy as jnp
from jax.experimental import pallas as pl


def kernel(x, edge_index, edge_attr, mW1, mb1, mW2, mb2, uW1, ub1, uW2, ub2, gamma, beta):
    raise NotImplementedError("write your pallas kernel here")


# speedup vs baseline: 3.6318x; 3.6318x over previous
import functools

import jax
import jax.numpy as jnp
from jax import lax
from jax.experimental import pallas as pl
from jax.experimental.pallas import tpu as pltpu
from jax.experimental.pallas import tpu_sc as plsc

N = 10000
E = 160000
D = 256
H = 128
NS = 16
NC = 2
EPT = E // NS
K = 80
NCHUNK = EPT // K
NP = 10240
ROWS_PT = NP // NS

_BN = 1000
_NB = N // _BN


def _precompute_body(x_ref, w_ref, b_ref, ab_ref):
    ab_ref[...] = (
        jnp.dot(x_ref[...], w_ref[...], preferred_element_type=jnp.float32)
        + b_ref[0]
    )


def _precompute(x, w1, bias2):
    return pl.pallas_call(
        _precompute_body,
        grid=(2, NC, _NB),
        in_specs=[
            pl.BlockSpec((_BN, D), lambda m, c, i: (i, 0)),
            pl.BlockSpec((D, H), lambda m, c, i: (m, c)),
            pl.BlockSpec((1, 1, H), lambda m, c, i: (m, 0, c)),
        ],
        out_specs=pl.BlockSpec((_BN, H), lambda m, c, i: (m * 2 * _NB + c * _NB + i, 0)),
        out_shape=jax.ShapeDtypeStruct((4 * N, H), jnp.float32),
    )(x, w1, bias2)


CB = NP // H


def _edge_body(src_hbm, dst_hbm, easp_hbm, ab_hbm, w_hbm, zero_hbm, zeroi_hbm,
               out_hbm, cnt_hbm,
               srcv, dstraw, dstadj, arows, brows, easp, wv, cntloc, ident,
               hacc, cntacc, sem_a, sem_b):
    c = lax.axis_index("c")
    s = lax.axis_index("s")

    pltpu.sync_copy(zero_hbm, hacc.at[pl.ds(s * ROWS_PT, ROWS_PT)])
    pltpu.sync_copy(w_hbm.at[pl.ds(c * H, H)], wv)

    @pl.when(s == 0)
    def _():
        pltpu.sync_copy(zeroi_hbm, cntacc)

    zeros_i = jnp.zeros((16,), jnp.int32)
    lanes = jax.lax.iota(jnp.int32, 16)

    @pl.loop(0, CB)
    def _zrow(r):
        for j in range(H // 16):
            cntloc[r, pl.ds(j * 16, 16)] = zeros_i

    for g in range(CB // 16):
        ident[pl.ds(g * 16, 16)] = lanes + (g * 16)

    wjs = [wv[pl.ds(j * 16, 16)] for j in range(H // 16)]
    aoff = jnp.broadcast_to(c * N, (16,)).astype(jnp.int32)
    boff = jnp.broadcast_to(2 * N + c * N, (16,)).astype(jnp.int32)

    plsc.subcore_barrier()

    @pl.loop(0, NCHUNK)
    def _chunk(k):
        base = s * EPT + k * K
        pltpu.sync_copy(src_hbm.at[pl.ds(base, K)], srcv)
        pltpu.sync_copy(dst_hbm.at[pl.ds(base, K)], dstraw)
        pltpu.sync_copy(easp_hbm.at[pl.ds(base, K)], easp)
        for j in range(K // 16):
            sl = pl.ds(j * 16, 16)
            d = dstraw[sl]
            srcv[sl] = srcv[sl] + aoff
            dstadj[sl] = d + boff
            cnts, last = plsc.scan_count(d)
            plsc.addupdate_scatter(
                cntloc, [jax.lax.shift_right_logical(d, 7), d & 127],
                cnts, mask=last)
        ca = pltpu.async_copy(ab_hbm.at[srcv], arows, sem_a)
        cb = pltpu.async_copy(ab_hbm.at[dstadj], brows, sem_b)
        ca.wait()
        cb.wait()

        @pl.loop(0, K)
        def _row(r):
            ev = easp[r, pl.ds(0, 16)]
            for j in range(H // 16):
                sl = pl.ds(j * 16, 16)
                arows[r, sl] = jnp.maximum(
                    arows[r, sl] + brows[r, sl] + ev * wjs[j], 0.0)

        pltpu.sync_copy(arows, hacc.at[dstraw], add=True)

    pltpu.sync_copy(cntloc, cntacc.at[ident], add=True)

    plsc.subcore_barrier()
    pltpu.sync_copy(
        hacc.at[pl.ds(s * ROWS_PT, ROWS_PT)],
        out_hbm.at[pl.ds(c * NP + s * ROWS_PT, ROWS_PT)],
    )

    @pl.when(s == 0)
    def _():
        pltpu.sync_copy(cntacc, cnt_hbm.at[pl.ds(c * CB, CB)])


@functools.cache
def _edge_kernel_fn():
    return pl.kernel(
        _edge_body,
        out_type=[
            jax.ShapeDtypeStruct((2 * NP, H), jnp.float32),
            jax.ShapeDtypeStruct((2 * CB, H), jnp.int32),
        ],
        mesh=plsc.VectorSubcoreMesh(
            core_axis_name="c", subcore_axis_name="s", num_cores=NC, num_subcores=NS
        ),
        compiler_params=pltpu.CompilerParams(needs_layout_passes=False),
        scratch_types=[
            pltpu.VMEM((K,), jnp.int32),
            pltpu.VMEM((K,), jnp.int32),
            pltpu.VMEM((K,), jnp.int32),
            pltpu.VMEM((K, H), jnp.float32),
            pltpu.VMEM((K, H), jnp.float32),
            pltpu.VMEM((K, 16), jnp.float32),
            pltpu.VMEM((H,), jnp.float32),
            pltpu.VMEM((CB, H), jnp.int32),
            pltpu.VMEM((CB,), jnp.int32),
            pltpu.VMEM_SHARED((NP, H), jnp.float32),
            pltpu.VMEM_SHARED((CB, H), jnp.int32),
            pltpu.SemaphoreType.DMA,
            pltpu.SemaphoreType.DMA,
        ],
    )


def _final_body(h0, h1, cref, a0, a1, b0, b1, x_ref, wrow, mW2, mb2,
                uW1a, uW1b, ub1, uW2, ub2, scale, beta, out_ref):
    a = jnp.concatenate([a0[...], a1[...]], axis=1)
    b = jnp.concatenate([b0[...], b1[...]], axis=1)
    hself = jnp.maximum(a + b + wrow[...], 0.0)
    hsum = jnp.concatenate([h0[...], h1[...]], axis=1) + hself
    cnt = cref[...].astype(jnp.float32) + 1.0
    aggr = (
        jnp.dot(hsum, mW2[...], preferred_element_type=jnp.float32) / cnt
        + mb2[...]
    )
    x = x_ref[...]
    u = jnp.maximum(
        jnp.dot(x, uW1a[...], preferred_element_type=jnp.float32)
        + jnp.dot(aggr, uW1b[...], preferred_element_type=jnp.float32)
        + ub1[...],
        0.0,
    )
    o = jnp.dot(u, uW2[...], preferred_element_type=jnp.float32) + ub2[...] + x
    out_ref[...] = jnp.maximum(o * scale[...] + beta[...], 0.0)


def _final(h0, h1, cnt, ab, x, wrow, mW2, mb2, uW1a, uW1b, ub1, uW2, ub2,
           scale, beta):
    full = lambda r, c: pl.BlockSpec((r, c), lambda i: (0, 0))
    return pl.pallas_call(
        _final_body,
        grid=(_NB,),
        in_specs=[
            pl.BlockSpec((_BN, H), lambda i: (i, 0)),
            pl.BlockSpec((_BN, H), lambda i: (i, 0)),
            pl.BlockSpec((_BN, 1), lambda i: (i, 0)),
            pl.BlockSpec((_BN, H), lambda i: (i, 0)),
            pl.BlockSpec((_BN, H), lambda i: (_NB + i, 0)),
            pl.BlockSpec((_BN, H), lambda i: (2 * _NB + i, 0)),
            pl.BlockSpec((_BN, H), lambda i: (3 * _NB + i, 0)),
            pl.BlockSpec((_BN, D), lambda i: (i, 0)),
            full(1, D), full(D, D), full(1, D),
            full(D, D), full(D, D), full(1, D),
            full(D, D), full(1, D), full(1, D), full(1, D),
        ],
        out_specs=pl.BlockSpec((_BN, D), lambda i: (i, 0)),
        out_shape=jax.ShapeDtypeStruct((N, D), jnp.float32),
    )(h0, h1, cnt, ab, ab, ab, ab, x, wrow, mW2, mb2,
      uW1a, uW1b, ub1, uW2, ub2, scale, beta)


def kernel(x, edge_index, edge_attr, mW1, mb1, mW2, mb2, uW1, ub1, uW2, ub2,
           gamma, beta):
    ei = edge_index.astype(jnp.int32)
    src = ei[0]
    dst = ei[1]
    easp = jnp.broadcast_to(edge_attr.astype(jnp.float32), (E, 16))
    w1 = mW1[: 2 * D]
    wrow = mW1[2 * D][None, :]
    bias2 = jnp.concatenate(
        [jnp.zeros((1, D), jnp.float32), mb1[None, :]], axis=0
    ).reshape(2, 1, D)
    zero = jnp.zeros((ROWS_PT, H), jnp.float32)
    zeroi = jnp.zeros((CB, H), jnp.int32)

    ab = _precompute(x, w1, bias2)
    hagg, cnt_out = _edge_kernel_fn()(src, dst, easp, ab, mW1[2 * D], zero, zeroi)
    h0 = hagg[:N]
    h1 = hagg[NP:NP + N]
    cnt = cnt_out[:CB].reshape(NP)[:N][:, None]
    scale = (gamma / jnp.sqrt(1.0 + 1e-5))[None, :]
    return _final(h0, h1, cnt, ab, x, wrow, mW2, mb2[None, :],
                  uW1[:D], uW1[D:], ub1[None, :], uW2, ub2[None, :],
                  scale, beta[None, :])


# speedup vs baseline: 3.9843x; 1.0971x over previous
import functools

import jax
import jax.numpy as jnp
from jax import lax
from jax.experimental import pallas as pl
from jax.experimental.pallas import tpu as pltpu
from jax.experimental.pallas import tpu_sc as plsc

N = 10000
E = 160000
D = 256
H = 128
NS = 16
NC = 2
K = 48
NCHUNK = 209
EPT = K * NCHUNK
EP = EPT * NS
NP = 10240
ROWS_PT = NP // NS

_BN = 1000
_NB = N // _BN


def _precompute_body(x_ref, w_ref, b_ref, ab_ref):
    ab_ref[...] = (
        jnp.dot(x_ref[...], w_ref[...], preferred_element_type=jnp.float32)
        + b_ref[0]
    )


def _precompute(x, w1, bias2):
    return pl.pallas_call(
        _precompute_body,
        grid=(2, NC, _NB),
        in_specs=[
            pl.BlockSpec((_BN, D), lambda m, c, i: (i, 0)),
            pl.BlockSpec((D, H), lambda m, c, i: (m, c)),
            pl.BlockSpec((1, 1, H), lambda m, c, i: (m, 0, c)),
        ],
        out_specs=pl.BlockSpec((_BN, H), lambda m, c, i: (m * 2 * _NB + c * _NB + i, 0)),
        out_shape=jax.ShapeDtypeStruct((4 * N, H), jnp.float32),
    )(x, w1, bias2)


CB = NP // H


def _edge_body(sadj_hbm, dadj_hbm, dst_hbm, easp_hbm, ab_hbm, w_hbm, zero_hbm,
               out_hbm,
               srcv0, dadjv0, draw0, easp0, ar0, br0,
               srcv1, dadjv1, draw1, easp1, ar1, br1,
               wv, hacc,
               semi0, sema0, semb0, semi1, semb1a, semb1b):
    c = lax.axis_index("c")
    s = lax.axis_index("s")
    bufs = [
        (srcv0, dadjv0, draw0, easp0, ar0, br0, semi0, sema0, semb0),
        (srcv1, dadjv1, draw1, easp1, ar1, br1, semi1, semb1a, semb1b),
    ]

    pltpu.sync_copy(zero_hbm, hacc.at[pl.ds(s * ROWS_PT, ROWS_PT)])
    pltpu.sync_copy(w_hbm.at[pl.ds(c * H, H)], wv)

    wjs = [wv[pl.ds(j * 16, 16)] for j in range(H // 16)]

    plsc.subcore_barrier()

    def fire_idx(b, k):
        srcv, dadjv, draw, easpv, *_rest, semi, _sa, _sb = bufs[b]
        base = s * EPT + k * K
        return [
            pltpu.async_copy(sadj_hbm.at[pl.ds(c * EP + base, K)], srcv, semi),
            pltpu.async_copy(dadj_hbm.at[pl.ds(c * EP + base, K)], dadjv, semi),
            pltpu.async_copy(dst_hbm.at[pl.ds(base, K)], draw, semi),
            pltpu.async_copy(easp_hbm.at[pl.ds(base, K)], easpv, semi),
        ]

    def fire_gather(b):
        srcv, dadjv, _draw, _easpv, ar, br, _si, sema, semb = bufs[b]
        pltpu.async_copy(ab_hbm.at[srcv], ar, sema)
        pltpu.async_copy(ab_hbm.at[dadjv], br, semb)

    def wait_gather(b):
        srcv, dadjv, _draw, _easpv, ar, br, _si, sema, semb = bufs[b]
        pltpu.make_async_copy(ab_hbm.at[srcv], ar, sema).wait()
        pltpu.make_async_copy(ab_hbm.at[dadjv], br, semb).wait()

    def compute_scatter(b):
        _srcv, _dadjv, draw, easpv, ar, br, *_sems = bufs[b]

        @pl.loop(0, K)
        def _row(r):
            ev = easpv[r, pl.ds(0, 16)]
            for j in range(H // 16):
                sl = pl.ds(j * 16, 16)
                ar[r, sl] = jnp.maximum(ar[r, sl] + br[r, sl] + ev * wjs[j], 0.0)

        pltpu.sync_copy(ar, hacc.at[draw], add=True)

    def phase(cur, k):
        nxt = 1 - cur
        descs = fire_idx(nxt, k + 1)
        wait_gather(cur)
        compute_scatter(cur)
        for d in descs:
            d.wait()
        fire_gather(nxt)

    for d in fire_idx(0, 0):
        d.wait()
    fire_gather(0)

    @pl.loop(0, (NCHUNK - 1) // 2)
    def _pair(g):
        phase(0, 2 * g)
        phase(1, 2 * g + 1)

    wait_gather(0)
    compute_scatter(0)

    plsc.subcore_barrier()
    pltpu.sync_copy(
        hacc.at[pl.ds(s * ROWS_PT, ROWS_PT)],
        out_hbm.at[pl.ds(c * NP + s * ROWS_PT, ROWS_PT)],
    )


@functools.cache
def _edge_kernel_fn():
    return pl.kernel(
        _edge_body,
        out_type=jax.ShapeDtypeStruct((2 * NP, H), jnp.float32),
        mesh=plsc.VectorSubcoreMesh(
            core_axis_name="c", subcore_axis_name="s", num_cores=NC, num_subcores=NS
        ),
        compiler_params=pltpu.CompilerParams(needs_layout_passes=False),
        scratch_types=(
            [
                pltpu.VMEM((K,), jnp.int32),
                pltpu.VMEM((K,), jnp.int32),
                pltpu.VMEM((K,), jnp.int32),
                pltpu.VMEM((K, 16), jnp.float32),
                pltpu.VMEM((K, H), jnp.float32),
                pltpu.VMEM((K, H), jnp.float32),
            ] * 2
            + [
                pltpu.VMEM((H,), jnp.float32),
                pltpu.VMEM_SHARED((NP, H), jnp.float32),
            ]
            + [pltpu.SemaphoreType.DMA] * 6
        ),
    )


ECT = EP // (NC * NS)
KC = 152


def _count_body(dst_hbm, zero_hbm, cnt_hbm, dbuf, ones_rows, cacc):
    c = lax.axis_index("c")
    s = lax.axis_index("s")
    pltpu.sync_copy(zero_hbm, cacc.at[pl.ds(s * ROWS_PT, ROWS_PT)])

    ones = jnp.full((16,), 1.0, jnp.float32)

    @pl.loop(0, KC)
    def _orow(r):
        for j in range(H // 16):
            ones_rows[r, pl.ds(j * 16, 16)] = ones

    plsc.subcore_barrier()
    base_t = (s * NC + c) * ECT

    @pl.loop(0, ECT // KC)
    def _chunk(k):
        pltpu.sync_copy(dst_hbm.at[pl.ds(base_t + k * KC, KC)], dbuf)
        pltpu.sync_copy(ones_rows, cacc.at[dbuf], add=True)

    plsc.subcore_barrier()
    pltpu.sync_copy(
        cacc.at[pl.ds(s * ROWS_PT, ROWS_PT)],
        cnt_hbm.at[pl.ds(c * NP + s * ROWS_PT, ROWS_PT)],
    )


@functools.cache
def _count_kernel_fn():
    return pl.kernel(
        _count_body,
        out_type=jax.ShapeDtypeStruct((2 * NP, H), jnp.float32),
        mesh=plsc.VectorSubcoreMesh(
            core_axis_name="c", subcore_axis_name="s", num_cores=NC, num_subcores=NS
        ),
        compiler_params=pltpu.CompilerParams(needs_layout_passes=False),
        scratch_types=[
            pltpu.VMEM((KC,), jnp.int32),
            pltpu.VMEM((KC, H), jnp.float32),
            pltpu.VMEM_SHARED((NP, H), jnp.float32),
        ],
    )


def _final_body(h0, h1, c0, c1, a0, a1, b0, b1, x_ref, wrow, mW2, mb2,
                uW1a, uW1b, ub1, uW2, ub2, scale, beta, out_ref):
    a = jnp.concatenate([a0[...], a1[...]], axis=1).astype(jnp.float32)
    b = jnp.concatenate([b0[...], b1[...]], axis=1).astype(jnp.float32)
    hself = jnp.maximum(a + b + wrow[...], 0.0)
    hsum = jnp.concatenate([h0[...], h1[...]], axis=1) + hself
    cnt = c0[...] + c1[...] + 1.0
    aggr = (
        jnp.dot(hsum, mW2[...], preferred_element_type=jnp.float32) / cnt
        + mb2[...]
    )
    x = x_ref[...]
    u = jnp.maximum(
        jnp.dot(x, uW1a[...], preferred_element_type=jnp.float32)
        + jnp.dot(aggr, uW1b[...], preferred_element_type=jnp.float32)
        + ub1[...],
        0.0,
    )
    o = jnp.dot(u, uW2[...], preferred_element_type=jnp.float32) + ub2[...] + x
    out_ref[...] = jnp.maximum(o * scale[...] + beta[...], 0.0)


def _final(h0, h1, c0, c1, ab, x, wrow, mW2, mb2, uW1a, uW1b, ub1, uW2, ub2,
           scale, beta):
    full = lambda r, c: pl.BlockSpec((r, c), lambda i: (0, 0))
    return pl.pallas_call(
        _final_body,
        grid=(_NB,),
        in_specs=[
            pl.BlockSpec((_BN, H), lambda i: (i, 0)),
            pl.BlockSpec((_BN, H), lambda i: (i, 0)),
            pl.BlockSpec((_BN, 1), lambda i: (i, 0)),
            pl.BlockSpec((_BN, 1), lambda i: (i, 0)),
            pl.BlockSpec((_BN, H), lambda i: (i, 0)),
            pl.BlockSpec((_BN, H), lambda i: (_NB + i, 0)),
            pl.BlockSpec((_BN, H), lambda i: (2 * _NB + i, 0)),
            pl.BlockSpec((_BN, H), lambda i: (3 * _NB + i, 0)),
            pl.BlockSpec((_BN, D), lambda i: (i, 0)),
            full(1, D), full(D, D), full(1, D),
            full(D, D), full(D, D), full(1, D),
            full(D, D), full(1, D), full(1, D), full(1, D),
        ],
        out_specs=pl.BlockSpec((_BN, D), lambda i: (i, 0)),
        out_shape=jax.ShapeDtypeStruct((N, D), jnp.float32),
    )(h0, h1, c0, c1, ab, ab, ab, ab, x, wrow, mW2, mb2,
      uW1a, uW1b, ub1, uW2, ub2, scale, beta)


def kernel(x, edge_index, edge_attr, mW1, mb1, mW2, mb2, uW1, ub1, uW2, ub2,
           gamma, beta):
    ei = edge_index.astype(jnp.int32)
    src = ei[0]
    dst = ei[1]
    pad = EP - E
    srcp = jnp.concatenate([src, jnp.zeros((pad,), jnp.int32)])
    dstz = jnp.concatenate([dst, jnp.zeros((pad,), jnp.int32)])
    dstp = jnp.concatenate([dst, jnp.full((pad,), N, jnp.int32)])
    sadj = jnp.concatenate([srcp, srcp + N])
    dadj = jnp.concatenate([dstz + 2 * N, dstz + 3 * N])
    easp = jnp.concatenate([
        jnp.broadcast_to(edge_attr.astype(jnp.float32), (E, 16)),
        jnp.zeros((pad, 16), jnp.float32)])
    w1 = mW1[: 2 * D]
    wrow = mW1[2 * D][None, :]
    bias2 = jnp.concatenate(
        [jnp.zeros((1, D), jnp.float32), mb1[None, :]], axis=0
    ).reshape(2, 1, D)
    zero = jnp.zeros((ROWS_PT, H), jnp.float32)

    ab = _precompute(x, w1, bias2)
    hagg = _edge_kernel_fn()(sadj, dadj, dstp, easp, ab, mW1[2 * D], zero)
    cnt_out = _count_kernel_fn()(dstp, zero)
    h0 = hagg[:N]
    h1 = hagg[NP:NP + N]
    c0 = cnt_out[:N, :1]
    c1 = cnt_out[NP:NP + N, :1]
    scale = (gamma / jnp.sqrt(1.0 + 1e-5))[None, :]
    return _final(h0, h1, c0, c1, ab, x, wrow, mW2, mb2[None, :],
                  uW1[:D], uW1[D:], ub1[None, :], uW2, ub2[None, :],
                  scale, beta[None, :])
